# baseline (speedup 1.0000x reference)
import jax
import jax.numpy as jnp
from jax.experimental import pallas as pl

_BC = 1024


def _body(s_ref, m_ref, o_ref):
    s = s_ref[...]
    neg = jnp.float32(-jnp.inf)
    for l in range(m_ref.shape[1]):
        ml = m_ref[:, l:l + 1]
        o_ref[l] = jnp.where(ml, neg, s)


def kernel(scores, M):
    B, C = scores.shape
    L = M.shape[0]
    sT = jnp.swapaxes(scores, 0, 1)
    mT = jnp.swapaxes(M, 0, 1)
    outT = pl.pallas_call(
        _body,
        grid=(B // _BC,),
        in_specs=[
            pl.BlockSpec((C, _BC), lambda j: (0, j)),
            pl.BlockSpec((C, L), lambda j: (0, 0)),
        ],
        out_specs=pl.BlockSpec((L, C, _BC), lambda j: (0, 0, j)),
        out_shape=jax.ShapeDtypeStruct((L, C, B), scores.dtype),
    )(sT, mT)
    return jnp.transpose(outT, (2, 0, 1))


# speedup vs baseline: 1.4802x; 1.4802x over previous
import jax
import jax.numpy as jnp
from jax.experimental import pallas as pl

_BC = 4096


def _body(s_ref, m_ref, o_ref):
    s = s_ref[...]
    neg = jnp.float32(-jnp.inf)
    for l in range(m_ref.shape[1]):
        ml = m_ref[:, l:l + 1]
        o_ref[l] = jnp.where(ml, neg, s)


def kernel(scores, M):
    B, C = scores.shape
    L = M.shape[0]
    sT = jnp.swapaxes(scores, 0, 1)
    mT = jnp.swapaxes(M, 0, 1)
    outT = pl.pallas_call(
        _body,
        grid=(B // _BC,),
        in_specs=[
            pl.BlockSpec((C, _BC), lambda j: (0, j)),
            pl.BlockSpec((C, L), lambda j: (0, 0)),
        ],
        out_specs=pl.BlockSpec((L, C, _BC), lambda j: (0, 0, j)),
        out_shape=jax.ShapeDtypeStruct((L, C, B), scores.dtype),
    )(sT, mT)
    return jnp.transpose(outT, (2, 0, 1))


# speedup vs baseline: 1.5734x; 1.0630x over previous
import jax
import jax.numpy as jnp
from jax.experimental import pallas as pl

_BC = 8192


def _body(s_ref, m_ref, o_ref):
    s = s_ref[...]
    neg = jnp.float32(-jnp.inf)
    for l in range(m_ref.shape[1]):
        ml = m_ref[:, l:l + 1]
        o_ref[l] = jnp.where(ml, neg, s)


def kernel(scores, M):
    B, C = scores.shape
    L = M.shape[0]
    sT = jnp.swapaxes(scores, 0, 1)
    mT = jnp.swapaxes(M, 0, 1)
    outT = pl.pallas_call(
        _body,
        grid=(B // _BC,),
        in_specs=[
            pl.BlockSpec((C, _BC), lambda j: (0, j)),
            pl.BlockSpec((C, L), lambda j: (0, 0)),
        ],
        out_specs=pl.BlockSpec((L, C, _BC), lambda j: (0, 0, j)),
        out_shape=jax.ShapeDtypeStruct((L, C, B), scores.dtype),
    )(sT, mT)
    return jnp.transpose(outT, (2, 0, 1))
